# Initial kernel scaffold; baseline (speedup 1.0000x reference)
import jax
import jax.numpy as jnp
from jax.experimental import pallas as pl


def kernel(x, edge_index, Wp, bp, W1, W2):
    raise NotImplementedError("write your pallas kernel here")


# speedup vs baseline: 3.1183x; 3.1183x over previous
import functools
import math

import jax
import jax.numpy as jnp
from jax import lax
from jax.experimental import pallas as pl
from jax.experimental.pallas import tpu as pltpu
from jax.experimental.pallas import tpu_sc as plsc

_N = 10000
_E = 160000
_D = 256
_NL = 4
_ALPHA = 0.5

_NT = 32
_K = 128
_KB = 1024
_EPAD = 163840
_NSC = _EPAD // _KB
_RANGE = 320
_ROWS = _NT * _RANGE
_CAP = _EPAD + _K
_TRASH = _RANGE

_BLK = 200
_NBLK = _N // _BLK

_sc_mesh = plsc.VectorSubcoreMesh(core_axis_name="c", subcore_axis_name="s")
_sc_params = pltpu.CompilerParams(needs_layout_passes=False)


@functools.partial(
    pl.kernel,
    out_type=(
        jax.ShapeDtypeStruct((_NT * _CAP,), jnp.int32),
        jax.ShapeDtypeStruct((_NT * _CAP,), jnp.int32),
        jax.ShapeDtypeStruct((_NT, 16), jnp.int32),
        jax.ShapeDtypeStruct((_ROWS, 16), jnp.float32),
    ),
    mesh=_sc_mesh,
    compiler_params=_sc_params,
    scratch_types=[
        pltpu.VMEM((_KB,), jnp.int32),
        pltpu.VMEM((_KB,), jnp.int32),
        pltpu.VMEM((272,), jnp.int32),
        pltpu.VMEM((272,), jnp.int32),
        pltpu.VMEM((16,), jnp.int32),
        pltpu.VMEM((_RANGE + 1, 16), jnp.float32),
    ],
)
def _bucketize(row_hbm, col_hbm, rows_out, lcs_out, cnt_out, deg_out,
               rbuf, cbuf, srow, slc, nvb, dacc):
    c = lax.axis_index("c")
    s = lax.axis_index("s")
    w = c * 16 + s
    obase = w * _CAP
    lane = lax.iota(jnp.int32, 16)
    lo = w * _RANGE

    def _scan(t, carry):
        nf, cnt = carry
        pltpu.sync_copy(row_hbm.at[pl.ds(t * _KB, _KB)], rbuf)
        pltpu.sync_copy(col_hbm.at[pl.ds(t * _KB, _KB)], cbuf)

        def _grp(u, carry2):
            nf2, cnt2 = carry2
            rv = rbuf[pl.ds(u * 16, 16)]
            cv = cbuf[pl.ds(u * 16, 16)] - lo
            m = (cv >= 0) & (cv < _RANGE)
            mi = m.astype(jnp.int32)
            rank = plsc.cumsum(mi) - 1
            nm = jnp.sum(mi)
            off = jnp.where(m, nf2 + rank, 256 + lane)
            plsc.store_scatter(srow, [off], rv)
            plsc.store_scatter(slc, [off], cv)
            nf3 = nf2 + nm

            def _flush():
                pltpu.sync_copy(srow.at[pl.ds(0, _K)],
                                rows_out.at[pl.ds(obase + cnt2 * _K, _K)])
                pltpu.sync_copy(slc.at[pl.ds(0, _K)],
                                lcs_out.at[pl.ds(obase + cnt2 * _K, _K)])
                srow[pl.ds(0, 16)] = srow[pl.ds(_K, 16)]
                slc[pl.ds(0, 16)] = slc[pl.ds(_K, 16)]

            pl.when(nf3 >= _K)(_flush)
            hit = nf3 >= _K
            return (jnp.where(hit, nf3 - _K, nf3),
                    jnp.where(hit, cnt2 + 1, cnt2))

        return lax.fori_loop(0, _KB // 16, _grp, (nf, cnt))

    nf, cnt = lax.fori_loop(0, _NSC, _scan, (jnp.int32(0), jnp.int32(0)))

    zero16 = jnp.zeros((16,), jnp.int32)
    trash16 = jnp.full((16,), _TRASH, jnp.int32)
    for t in range(9):
        plsc.store_scatter(srow, [nf + t * 16 + lane], zero16)
        plsc.store_scatter(slc, [nf + t * 16 + lane], trash16)
    pltpu.sync_copy(srow.at[pl.ds(0, _K)],
                    rows_out.at[pl.ds(obase + cnt * _K, _K)])
    pltpu.sync_copy(slc.at[pl.ds(0, _K)],
                    lcs_out.at[pl.ds(obase + cnt * _K, _K)])
    nch = cnt + jnp.where(nf > 0, 1, 0).astype(jnp.int32)
    nvb[...] = jnp.broadcast_to(nch, (16,)).astype(jnp.int32)
    pltpu.sync_copy(nvb, cnt_out.at[w])

    fz = jnp.zeros((16,), jnp.float32)

    def _dz(i, carry):
        dacc[i, :] = fz
        return carry

    lax.fori_loop(0, _RANGE + 1, _dz, 0)
    one16 = jnp.ones((16,), jnp.float32)

    def _dchunk(j, carry):
        pltpu.sync_copy(lcs_out.at[pl.ds(obase + j * _K, _K)],
                        cbuf.at[pl.ds(0, _K)])
        for u in range(_K // 16):
            lcv = cbuf[pl.ds(u * 16, 16)]
            for e2 in range(16):
                lc = jnp.sum(jnp.where(lane == e2, lcv, 0))
                plsc.addupdate(dacc.at[lc], one16)
        return carry

    lax.fori_loop(0, nch, _dchunk, 0)
    pltpu.sync_copy(dacc.at[pl.ds(0, _RANGE)],
                    deg_out.at[pl.ds(w * _RANGE, _RANGE)])


@functools.partial(
    pl.kernel,
    out_type=jax.ShapeDtypeStruct((_ROWS, _D), jnp.float32),
    mesh=_sc_mesh,
    compiler_params=_sc_params,
    scratch_types=[
        pltpu.VMEM((16,), jnp.int32),
        pltpu.VMEM((_K,), jnp.int32),
        pltpu.VMEM((_K,), jnp.int32),
        pltpu.VMEM((_K, _D), jnp.float32),
        pltpu.VMEM((_RANGE + 1, _D), jnp.float32),
        pltpu.SemaphoreType.DMA,
    ],
)
def _propagate(rows_hbm, lcs_hbm, cnt_hbm, g_hbm, s_out,
               nv, ridx, lcb, buf, acc, sem):
    c = lax.axis_index("c")
    s = lax.axis_index("s")
    w = c * 16 + s
    obase = w * _CAP
    lane = lax.iota(jnp.int32, 16)

    fz = jnp.zeros((16,), jnp.float32)

    def _az(i, carry):
        for v in range(_D // 16):
            acc[i, pl.ds(v * 16, 16)] = fz
        return carry

    lax.fori_loop(0, _RANGE + 1, _az, 0)

    pltpu.sync_copy(cnt_hbm.at[w], nv)
    n = jnp.sum(jnp.where(lane == 0, nv[...], 0))

    def _chunk(j, carry):
        base = obase + j * _K
        pltpu.sync_copy(rows_hbm.at[pl.ds(base, _K)], ridx)
        pltpu.sync_copy(lcs_hbm.at[pl.ds(base, _K)], lcb)
        pltpu.async_copy(g_hbm.at[ridx], buf, sem).wait()
        for u in range(_K // 16):
            lcv = lcb[pl.ds(u * 16, 16)]
            for e2 in range(16):
                lc = jnp.sum(jnp.where(lane == e2, lcv, 0))
                e = u * 16 + e2
                for v in range(_D // 16):
                    plsc.addupdate(acc.at[lc, pl.ds(v * 16, 16)],
                                   buf[e, pl.ds(v * 16, 16)])
        return carry

    lax.fori_loop(0, n, _chunk, 0)
    pltpu.sync_copy(acc.at[pl.ds(0, _RANGE)],
                    s_out.at[pl.ds(w * _RANGE, _RANGE)])


def _prologue_body(x_ref, wp_ref, bp_ref, w2_ref, deg_ref,
                   g_ref, c_ref, dis_ref):
    h0 = jnp.dot(x_ref[...], wp_ref[...],
                 preferred_element_type=jnp.float32) + bp_ref[...]
    dis = lax.rsqrt(deg_ref[...][:, 0:1] + 1.0)
    g_ref[...] = dis * h0
    dis_ref[...] = jnp.broadcast_to(dis, (_BLK, 128))
    for i in range(_NL):
        beta = math.log(1.0 / (i + 1) + 1.0)
        c_ref[i, :, :] = (_ALPHA * (1.0 - beta)) * h0 + (_ALPHA * beta) * jnp.dot(
            h0, w2_ref[i], preferred_element_type=jnp.float32)


_prologue = pl.pallas_call(
    _prologue_body,
    grid=(_NBLK,),
    in_specs=[
        pl.BlockSpec((_BLK, _D), lambda j: (j, 0)),
        pl.BlockSpec((_D, _D), lambda j: (0, 0)),
        pl.BlockSpec((1, _D), lambda j: (0, 0)),
        pl.BlockSpec((_NL, _D, _D), lambda j: (0, 0, 0)),
        pl.BlockSpec((_BLK, 16), lambda j: (j, 0)),
    ],
    out_specs=[
        pl.BlockSpec((_BLK, _D), lambda j: (j, 0)),
        pl.BlockSpec((_NL, _BLK, _D), lambda j: (0, j, 0)),
        pl.BlockSpec((_BLK, 128), lambda j: (j, 0)),
    ],
    out_shape=[
        jax.ShapeDtypeStruct((_N, _D), jnp.float32),
        jax.ShapeDtypeStruct((_NL, _N, _D), jnp.float32),
        jax.ShapeDtypeStruct((_N, 128), jnp.float32),
    ],
)


def _layer_body(beta, is_last, s_ref, g_ref, cc_ref, dis_ref, w1_ref, o_ref):
    dis = dis_ref[...][:, 0:1]
    xp = dis * (s_ref[...] + g_ref[...])
    x1 = _ALPHA * xp
    out = cc_ref[...] + (1.0 - beta) * x1 + beta * jnp.dot(
        x1, w1_ref[...], preferred_element_type=jnp.float32)
    h = jnp.maximum(out, 0.0)
    o_ref[...] = h if is_last else dis * h


def _make_layer(i):
    beta = math.log(1.0 / (i + 1) + 1.0)
    return pl.pallas_call(
        functools.partial(_layer_body, beta, i == _NL - 1),
        grid=(_NBLK,),
        in_specs=[
            pl.BlockSpec((_BLK, _D), lambda j: (j, 0)),
            pl.BlockSpec((_BLK, _D), lambda j: (j, 0)),
            pl.BlockSpec((_BLK, _D), lambda j: (j, 0)),
            pl.BlockSpec((_BLK, 128), lambda j: (j, 0)),
            pl.BlockSpec((_D, _D), lambda j: (0, 0)),
        ],
        out_specs=pl.BlockSpec((_BLK, _D), lambda j: (j, 0)),
        out_shape=jax.ShapeDtypeStruct((_N, _D), jnp.float32),
    )


_layers = [_make_layer(i) for i in range(_NL)]


def kernel(x, edge_index, Wp, bp, W1, W2):
    row = edge_index[0]
    col = edge_index[1]
    pad = _EPAD - _E
    rowp = jnp.concatenate([row, jnp.zeros((pad,), row.dtype)])
    colp = jnp.concatenate([col, jnp.full((pad,), _ROWS, col.dtype)])

    brows, blcs, bcnt, deg = _bucketize(rowp, colp)
    g, C, dis = _prologue(x, Wp, bp.reshape(1, _D), W2, deg[:_N])

    for i in range(_NL):
        spart = _propagate(brows, blcs, bcnt, g)
        g = _layers[i](spart, g, C[i], dis, W1[i])
    return g


# speedup vs baseline: 3.4753x; 1.1145x over previous
import functools
import math

import jax
import jax.numpy as jnp
from jax import lax
from jax.experimental import pallas as pl
from jax.experimental.pallas import tpu as pltpu
from jax.experimental.pallas import tpu_sc as plsc

_N = 10000
_E = 160000
_D = 256
_NL = 4
_ALPHA = 0.5

_NT = 32
_K = 128
_KB = 1024
_EPAD = 163840
_NSC = _EPAD // _KB
_RANGE = 320
_ROWS = _NT * _RANGE
_B = 64
_CAPB = _EPAD // _B + 4
_CAPW = _CAPB * 128
_TRASH = _RANGE

_BLK = 200
_NBLK = _N // _BLK

_sc_mesh = plsc.VectorSubcoreMesh(core_axis_name="c", subcore_axis_name="s")
_sc_params = pltpu.CompilerParams(needs_layout_passes=False)


@functools.partial(
    pl.kernel,
    out_type=(
        jax.ShapeDtypeStruct((_NT * _CAPW,), jnp.int32),
        jax.ShapeDtypeStruct((_NT, 16), jnp.int32),
        jax.ShapeDtypeStruct((_ROWS, 16), jnp.float32),
    ),
    mesh=_sc_mesh,
    compiler_params=_sc_params,
    scratch_types=[
        pltpu.VMEM((_KB,), jnp.int32),
        pltpu.VMEM((_KB,), jnp.int32),
        pltpu.VMEM((144,), jnp.int32),
        pltpu.VMEM((144,), jnp.int32),
        pltpu.VMEM((16,), jnp.int32),
        pltpu.VMEM((_RANGE + 1, 16), jnp.float32),
    ],
)
def _bucketize(row_hbm, col_hbm, bidx_out, cnt_out, deg_out,
               rbuf, cbuf, srow, slc, nvb, dacc):
    c = lax.axis_index("c")
    s = lax.axis_index("s")
    w = c * 16 + s
    obase = w * _CAPW
    lane = lax.iota(jnp.int32, 16)
    lo = w * _RANGE

    def _scan(t, carry):
        nf, cnt = carry
        pltpu.sync_copy(row_hbm.at[pl.ds(t * _KB, _KB)], rbuf)
        pltpu.sync_copy(col_hbm.at[pl.ds(t * _KB, _KB)], cbuf)

        def _grp(u, carry2):
            nf2, cnt2 = carry2
            rv = rbuf[pl.ds(u * 16, 16)]
            cv = cbuf[pl.ds(u * 16, 16)] - lo
            m = (cv >= 0) & (cv < _RANGE)
            mi = m.astype(jnp.int32)
            rank = plsc.cumsum(mi) - 1
            nm = jnp.sum(mi)
            off = jnp.where(m, nf2 + rank, 128 + lane)
            plsc.store_scatter(srow, [off], rv)
            plsc.store_scatter(slc, [off], cv)
            nf3 = nf2 + nm

            def _flush():
                base = obase + cnt2 * 128
                pltpu.sync_copy(srow.at[pl.ds(0, _B)],
                                bidx_out.at[pl.ds(base, _B)])
                pltpu.sync_copy(slc.at[pl.ds(0, _B)],
                                bidx_out.at[pl.ds(base + _B, _B)])
                srow[pl.ds(0, 16)] = srow[pl.ds(_B, 16)]
                slc[pl.ds(0, 16)] = slc[pl.ds(_B, 16)]

            pl.when(nf3 >= _B)(_flush)
            hit = nf3 >= _B
            return (jnp.where(hit, nf3 - _B, nf3),
                    jnp.where(hit, cnt2 + 1, cnt2))

        return lax.fori_loop(0, _KB // 16, _grp, (nf, cnt))

    nf, cnt = lax.fori_loop(0, _NSC, _scan, (jnp.int32(0), jnp.int32(0)))

    zero16 = jnp.zeros((16,), jnp.int32)
    trash16 = jnp.full((16,), _TRASH, jnp.int32)
    for t in range(5):
        plsc.store_scatter(srow, [nf + t * 16 + lane], zero16)
        plsc.store_scatter(slc, [nf + t * 16 + lane], trash16)
    base = obase + cnt * 128
    pltpu.sync_copy(srow.at[pl.ds(0, _B)], bidx_out.at[pl.ds(base, _B)])
    pltpu.sync_copy(slc.at[pl.ds(0, _B)], bidx_out.at[pl.ds(base + _B, _B)])
    nch = cnt + jnp.where(nf > 0, 1, 0).astype(jnp.int32)
    nvb[...] = jnp.broadcast_to(nch, (16,)).astype(jnp.int32)
    pltpu.sync_copy(nvb, cnt_out.at[w])

    fz = jnp.zeros((16,), jnp.float32)

    def _dz(i, carry):
        dacc[i, :] = fz
        return carry

    lax.fori_loop(0, _RANGE + 1, _dz, 0)
    one16 = jnp.ones((16,), jnp.float32)

    def _dchunk(j, carry):
        pltpu.sync_copy(bidx_out.at[pl.ds(obase + j * 128 + _B, _B)],
                        cbuf.at[pl.ds(0, _B)])
        for u in range(_B // 16):
            lcv = cbuf[pl.ds(u * 16, 16)]
            for e2 in range(16):
                lc = jnp.sum(jnp.where(lane == e2, lcv, 0))
                plsc.addupdate(dacc.at[lc], one16)
        return carry

    lax.fori_loop(0, nch, _dchunk, 0)
    pltpu.sync_copy(dacc.at[pl.ds(0, _RANGE)],
                    deg_out.at[pl.ds(w * _RANGE, _RANGE)])


@functools.partial(
    pl.kernel,
    out_type=jax.ShapeDtypeStruct((_ROWS, _D), jnp.float32),
    mesh=_sc_mesh,
    compiler_params=_sc_params,
    scratch_types=[
        pltpu.VMEM((16,), jnp.int32),
        pltpu.VMEM((128,), jnp.int32),
        pltpu.VMEM((128,), jnp.int32),
        pltpu.VMEM((_B, _D), jnp.float32),
        pltpu.VMEM((_B, _D), jnp.float32),
        pltpu.VMEM((_RANGE + 1, _D), jnp.float32),
        pltpu.SemaphoreType.DMA,
        pltpu.SemaphoreType.DMA,
    ],
)
def _propagate(bidx_hbm, cnt_hbm, g_hbm, s_out,
               nv, ib0, ib1, buf0, buf1, acc, sem0, sem1):
    c = lax.axis_index("c")
    s = lax.axis_index("s")
    w = c * 16 + s
    obase = w * _CAPW
    lane = lax.iota(jnp.int32, 16)

    pltpu.sync_copy(cnt_hbm.at[w], nv)
    n = jnp.sum(jnp.where(lane == 0, nv[...], 0))

    def _idx(j, ib):
        pltpu.sync_copy(bidx_hbm.at[pl.ds(obase + j * 128, 128)], ib)

    def _fire(ib, buf, sem):
        pltpu.async_copy(g_hbm.at[ib.at[pl.ds(0, _B)]], buf, sem)

    def _wait(ib, buf, sem):
        pltpu.make_async_copy(g_hbm.at[ib.at[pl.ds(0, _B)]], buf, sem).wait()

    def _compute(ib, buf):
        for u in range(_B // 16):
            lcv = ib[pl.ds(_B + u * 16, 16)]
            for e2 in range(16):
                lc = jnp.sum(jnp.where(lane == e2, lcv, 0))
                e = u * 16 + e2
                for v in range(_D // 16):
                    plsc.addupdate(acc.at[lc, pl.ds(v * 16, 16)],
                                   buf[e, pl.ds(v * 16, 16)])

    @pl.when(n > 0)
    def _prime():
        _idx(0, ib0)
        _fire(ib0, buf0, sem0)

    fz = jnp.zeros((16,), jnp.float32)

    def _az(i, carry):
        for v in range(_D // 16):
            acc[i, pl.ds(v * 16, 16)] = fz
        return carry

    lax.fori_loop(0, _RANGE + 1, _az, 0)

    def _body(jj, carry):
        j0 = jj * 2
        j1 = j0 + 1

        @pl.when(j1 < n)
        def _():
            _idx(j1, ib1)
            _fire(ib1, buf1, sem1)

        _wait(ib0, buf0, sem0)
        _compute(ib0, buf0)

        @pl.when(j0 + 2 < n)
        def _():
            _idx(j0 + 2, ib0)
            _fire(ib0, buf0, sem0)

        @pl.when(j1 < n)
        def _():
            _wait(ib1, buf1, sem1)
            _compute(ib1, buf1)

        return carry

    lax.fori_loop(0, (n + 1) // 2, _body, 0)
    pltpu.sync_copy(acc.at[pl.ds(0, _RANGE)],
                    s_out.at[pl.ds(w * _RANGE, _RANGE)])


def _prologue_body(x_ref, wp_ref, bp_ref, w2_ref, deg_ref,
                   g_ref, c_ref, dis_ref):
    h0 = jnp.dot(x_ref[...], wp_ref[...],
                 preferred_element_type=jnp.float32) + bp_ref[...]
    dis = lax.rsqrt(deg_ref[...][:, 0:1] + 1.0)
    g_ref[...] = dis * h0
    dis_ref[...] = jnp.broadcast_to(dis, (_BLK, 128))
    for i in range(_NL):
        beta = math.log(1.0 / (i + 1) + 1.0)
        c_ref[i, :, :] = (_ALPHA * (1.0 - beta)) * h0 + (_ALPHA * beta) * jnp.dot(
            h0, w2_ref[i], preferred_element_type=jnp.float32)


_prologue = pl.pallas_call(
    _prologue_body,
    grid=(_NBLK,),
    in_specs=[
        pl.BlockSpec((_BLK, _D), lambda j: (j, 0)),
        pl.BlockSpec((_D, _D), lambda j: (0, 0)),
        pl.BlockSpec((1, _D), lambda j: (0, 0)),
        pl.BlockSpec((_NL, _D, _D), lambda j: (0, 0, 0)),
        pl.BlockSpec((_BLK, 16), lambda j: (j, 0)),
    ],
    out_specs=[
        pl.BlockSpec((_BLK, _D), lambda j: (j, 0)),
        pl.BlockSpec((_NL, _BLK, _D), lambda j: (0, j, 0)),
        pl.BlockSpec((_BLK, 128), lambda j: (j, 0)),
    ],
    out_shape=[
        jax.ShapeDtypeStruct((_N, _D), jnp.float32),
        jax.ShapeDtypeStruct((_NL, _N, _D), jnp.float32),
        jax.ShapeDtypeStruct((_N, 128), jnp.float32),
    ],
)


def _layer_body(beta, is_last, s_ref, g_ref, cc_ref, dis_ref, w1_ref, o_ref):
    dis = dis_ref[...][:, 0:1]
    xp = dis * (s_ref[...] + g_ref[...])
    x1 = _ALPHA * xp
    out = cc_ref[...] + (1.0 - beta) * x1 + beta * jnp.dot(
        x1, w1_ref[...], preferred_element_type=jnp.float32)
    h = jnp.maximum(out, 0.0)
    o_ref[...] = h if is_last else dis * h


def _make_layer(i):
    beta = math.log(1.0 / (i + 1) + 1.0)
    return pl.pallas_call(
        functools.partial(_layer_body, beta, i == _NL - 1),
        grid=(_NBLK,),
        in_specs=[
            pl.BlockSpec((_BLK, _D), lambda j: (j, 0)),
            pl.BlockSpec((_BLK, _D), lambda j: (j, 0)),
            pl.BlockSpec((_BLK, _D), lambda j: (j, 0)),
            pl.BlockSpec((_BLK, 128), lambda j: (j, 0)),
            pl.BlockSpec((_D, _D), lambda j: (0, 0)),
        ],
        out_specs=pl.BlockSpec((_BLK, _D), lambda j: (j, 0)),
        out_shape=jax.ShapeDtypeStruct((_N, _D), jnp.float32),
    )


_layers = [_make_layer(i) for i in range(_NL)]


def kernel(x, edge_index, Wp, bp, W1, W2):
    row = edge_index[0]
    col = edge_index[1]
    pad = _EPAD - _E
    rowp = jnp.concatenate([row, jnp.zeros((pad,), row.dtype)])
    colp = jnp.concatenate([col, jnp.full((pad,), _ROWS, col.dtype)])

    bidx, bcnt, deg = _bucketize(rowp, colp)
    g, C, dis = _prologue(x, Wp, bp.reshape(1, _D), W2, deg[:_N])

    for i in range(_NL):
        spart = _propagate(bidx, bcnt, g)
        g = _layers[i](spart, g, C[i], dis, W1[i])
    return g


# speedup vs baseline: 3.5438x; 1.0197x over previous
import functools
import math

import jax
import jax.numpy as jnp
from jax import lax
from jax.experimental import pallas as pl
from jax.experimental.pallas import tpu as pltpu
from jax.experimental.pallas import tpu_sc as plsc

_N = 10000
_E = 160000
_D = 256
_NL = 4
_ALPHA = 0.5

_NT = 32
_K = 128
_KB = 1024
_EPAD = 163840
_NSC = _EPAD // _KB
_RANGE = 320
_ROWS = _NT * _RANGE
_B = 64
_CAPB = _EPAD // _B + 4
_CAPW = _CAPB * 128
_TRASH = _RANGE

_BLK = 200
_NBLK = _N // _BLK

_sc_mesh = plsc.VectorSubcoreMesh(core_axis_name="c", subcore_axis_name="s")
_sc_params = pltpu.CompilerParams(needs_layout_passes=False)


@functools.partial(
    pl.kernel,
    out_type=(
        jax.ShapeDtypeStruct((_NT * _CAPW,), jnp.int32),
        jax.ShapeDtypeStruct((_NT, 16), jnp.int32),
        jax.ShapeDtypeStruct((_ROWS, 16), jnp.float32),
    ),
    mesh=_sc_mesh,
    compiler_params=_sc_params,
    scratch_types=[
        pltpu.VMEM((_KB,), jnp.int32),
        pltpu.VMEM((_KB,), jnp.int32),
        pltpu.VMEM((144,), jnp.int32),
        pltpu.VMEM((144,), jnp.int32),
        pltpu.VMEM((16,), jnp.int32),
        pltpu.VMEM((_RANGE + 1, 16), jnp.float32),
    ],
)
def _bucketize(row_hbm, col_hbm, bidx_out, cnt_out, deg_out,
               rbuf, cbuf, srow, slc, nvb, dacc):
    c = lax.axis_index("c")
    s = lax.axis_index("s")
    w = c * 16 + s
    obase = w * _CAPW
    lane = lax.iota(jnp.int32, 16)
    lo = w * _RANGE

    def _scan(t, carry):
        nf, cnt = carry
        pltpu.sync_copy(row_hbm.at[pl.ds(t * _KB, _KB)], rbuf)
        pltpu.sync_copy(col_hbm.at[pl.ds(t * _KB, _KB)], cbuf)

        def _grp(u, carry2):
            nf2, cnt2 = carry2
            rv = rbuf[pl.ds(u * 16, 16)]
            cv = cbuf[pl.ds(u * 16, 16)] - lo
            m = (cv >= 0) & (cv < _RANGE)
            mi = m.astype(jnp.int32)
            rank = plsc.cumsum(mi) - 1
            nm = jnp.sum(mi)
            off = jnp.where(m, nf2 + rank, 128 + lane)
            plsc.store_scatter(srow, [off], rv)
            plsc.store_scatter(slc, [off], cv)
            nf3 = nf2 + nm

            def _flush():
                base = obase + cnt2 * 128
                pltpu.sync_copy(srow.at[pl.ds(0, _B)],
                                bidx_out.at[pl.ds(base, _B)])
                pltpu.sync_copy(slc.at[pl.ds(0, _B)],
                                bidx_out.at[pl.ds(base + _B, _B)])
                srow[pl.ds(0, 16)] = srow[pl.ds(_B, 16)]
                slc[pl.ds(0, 16)] = slc[pl.ds(_B, 16)]

            pl.when(nf3 >= _B)(_flush)
            hit = nf3 >= _B
            return (jnp.where(hit, nf3 - _B, nf3),
                    jnp.where(hit, cnt2 + 1, cnt2))

        return lax.fori_loop(0, _KB // 16, _grp, (nf, cnt))

    nf, cnt = lax.fori_loop(0, _NSC, _scan, (jnp.int32(0), jnp.int32(0)))

    zero16 = jnp.zeros((16,), jnp.int32)
    trash16 = jnp.full((16,), _TRASH, jnp.int32)
    for t in range(5):
        plsc.store_scatter(srow, [nf + t * 16 + lane], zero16)
        plsc.store_scatter(slc, [nf + t * 16 + lane], trash16)
    base = obase + cnt * 128
    pltpu.sync_copy(srow.at[pl.ds(0, _B)], bidx_out.at[pl.ds(base, _B)])
    pltpu.sync_copy(slc.at[pl.ds(0, _B)], bidx_out.at[pl.ds(base + _B, _B)])
    nch = cnt + jnp.where(nf > 0, 1, 0).astype(jnp.int32)
    nvb[...] = jnp.broadcast_to(nch, (16,)).astype(jnp.int32)
    pltpu.sync_copy(nvb, cnt_out.at[w])

    fz = jnp.zeros((16,), jnp.float32)

    def _dz(i, carry):
        dacc[i, :] = fz
        return carry

    lax.fori_loop(0, _RANGE + 1, _dz, 0)
    one16 = jnp.ones((16,), jnp.float32)

    def _dchunk(j, carry):
        pltpu.sync_copy(bidx_out.at[pl.ds(obase + j * 128 + _B, _B)],
                        cbuf.at[pl.ds(0, _B)])
        for u in range(_B // 16):
            lcv = cbuf[pl.ds(u * 16, 16)]
            for e2 in range(16):
                lc = jnp.sum(jnp.where(lane == e2, lcv, 0))
                plsc.addupdate(dacc.at[lc], one16)
        return carry

    lax.fori_loop(0, nch, _dchunk, 0)
    pltpu.sync_copy(dacc.at[pl.ds(0, _RANGE)],
                    deg_out.at[pl.ds(w * _RANGE, _RANGE)])


@functools.partial(
    pl.kernel,
    out_type=jax.ShapeDtypeStruct((_ROWS, _D), jnp.float32),
    mesh=_sc_mesh,
    compiler_params=_sc_params,
    scratch_types=[
        pltpu.VMEM((16,), jnp.int32),
        pltpu.VMEM((128,), jnp.int32),
        pltpu.VMEM((128,), jnp.int32),
        pltpu.VMEM((_B, _D), jnp.float32),
        pltpu.VMEM((_B, _D), jnp.float32),
        pltpu.VMEM((_RANGE + 1, _D), jnp.float32),
        pltpu.SMEM((16,), jnp.int32),
        pltpu.SemaphoreType.DMA,
        pltpu.SemaphoreType.DMA,
    ],
)
def _propagate(bidx_hbm, cnt_hbm, g_hbm, s_out,
               nv, ib0, ib1, buf0, buf1, acc, lcsm, sem0, sem1):
    c = lax.axis_index("c")
    s = lax.axis_index("s")
    w = c * 16 + s
    obase = w * _CAPW
    lane = lax.iota(jnp.int32, 16)

    pltpu.sync_copy(cnt_hbm.at[w], nv)
    n = jnp.sum(jnp.where(lane == 0, nv[...], 0))

    def _idx(j, ib):
        pltpu.sync_copy(bidx_hbm.at[pl.ds(obase + j * 128, 128)], ib)

    def _fire(ib, buf, sem):
        pltpu.async_copy(g_hbm.at[ib.at[pl.ds(0, _B)]], buf, sem)

    def _wait(ib, buf, sem):
        pltpu.make_async_copy(g_hbm.at[ib.at[pl.ds(0, _B)]], buf, sem).wait()

    def _compute(ib, buf):
        for u in range(_B // 16):
            lcv = ib[pl.ds(_B + u * 16, 16)]
            for e2 in range(16):
                lcsm[e2] = jnp.sum(jnp.where(lane == e2, lcv, 0))
            for e2 in range(16):
                lc = lcsm[e2]
                e = u * 16 + e2
                for v in range(_D // 16):
                    plsc.addupdate(acc.at[lc, pl.ds(v * 16, 16)],
                                   buf[e, pl.ds(v * 16, 16)])

    @pl.when(n > 0)
    def _prime():
        _idx(0, ib0)
        _fire(ib0, buf0, sem0)

    fz = jnp.zeros((16,), jnp.float32)

    def _az(i, carry):
        for v in range(_D // 16):
            acc[i, pl.ds(v * 16, 16)] = fz
        return carry

    lax.fori_loop(0, _RANGE + 1, _az, 0)

    def _body(jj, carry):
        j0 = jj * 2
        j1 = j0 + 1

        @pl.when(j1 < n)
        def _():
            _idx(j1, ib1)
            _fire(ib1, buf1, sem1)

        _wait(ib0, buf0, sem0)
        _compute(ib0, buf0)

        @pl.when(j0 + 2 < n)
        def _():
            _idx(j0 + 2, ib0)
            _fire(ib0, buf0, sem0)

        @pl.when(j1 < n)
        def _():
            _wait(ib1, buf1, sem1)
            _compute(ib1, buf1)

        return carry

    lax.fori_loop(0, (n + 1) // 2, _body, 0)
    pltpu.sync_copy(acc.at[pl.ds(0, _RANGE)],
                    s_out.at[pl.ds(w * _RANGE, _RANGE)])


def _prologue_body(x_ref, wp_ref, bp_ref, w2_ref, deg_ref,
                   g_ref, c_ref, dis_ref):
    h0 = jnp.dot(x_ref[...], wp_ref[...],
                 preferred_element_type=jnp.float32) + bp_ref[...]
    dis = lax.rsqrt(deg_ref[...][:, 0:1] + 1.0)
    g_ref[...] = dis * h0
    dis_ref[...] = jnp.broadcast_to(dis, (_BLK, 128))
    for i in range(_NL):
        beta = math.log(1.0 / (i + 1) + 1.0)
        c_ref[i, :, :] = (_ALPHA * (1.0 - beta)) * h0 + (_ALPHA * beta) * jnp.dot(
            h0, w2_ref[i], preferred_element_type=jnp.float32)


_prologue = pl.pallas_call(
    _prologue_body,
    grid=(_NBLK,),
    in_specs=[
        pl.BlockSpec((_BLK, _D), lambda j: (j, 0)),
        pl.BlockSpec((_D, _D), lambda j: (0, 0)),
        pl.BlockSpec((1, _D), lambda j: (0, 0)),
        pl.BlockSpec((_NL, _D, _D), lambda j: (0, 0, 0)),
        pl.BlockSpec((_BLK, 16), lambda j: (j, 0)),
    ],
    out_specs=[
        pl.BlockSpec((_BLK, _D), lambda j: (j, 0)),
        pl.BlockSpec((_NL, _BLK, _D), lambda j: (0, j, 0)),
        pl.BlockSpec((_BLK, 128), lambda j: (j, 0)),
    ],
    out_shape=[
        jax.ShapeDtypeStruct((_N, _D), jnp.float32),
        jax.ShapeDtypeStruct((_NL, _N, _D), jnp.float32),
        jax.ShapeDtypeStruct((_N, 128), jnp.float32),
    ],
)


def _layer_body(beta, is_last, s_ref, g_ref, cc_ref, dis_ref, w1_ref, o_ref):
    dis = dis_ref[...][:, 0:1]
    xp = dis * (s_ref[...] + g_ref[...])
    x1 = _ALPHA * xp
    out = cc_ref[...] + (1.0 - beta) * x1 + beta * jnp.dot(
        x1, w1_ref[...], preferred_element_type=jnp.float32)
    h = jnp.maximum(out, 0.0)
    o_ref[...] = h if is_last else dis * h


def _make_layer(i):
    beta = math.log(1.0 / (i + 1) + 1.0)
    return pl.pallas_call(
        functools.partial(_layer_body, beta, i == _NL - 1),
        grid=(_NBLK,),
        in_specs=[
            pl.BlockSpec((_BLK, _D), lambda j: (j, 0)),
            pl.BlockSpec((_BLK, _D), lambda j: (j, 0)),
            pl.BlockSpec((_BLK, _D), lambda j: (j, 0)),
            pl.BlockSpec((_BLK, 128), lambda j: (j, 0)),
            pl.BlockSpec((_D, _D), lambda j: (0, 0)),
        ],
        out_specs=pl.BlockSpec((_BLK, _D), lambda j: (j, 0)),
        out_shape=jax.ShapeDtypeStruct((_N, _D), jnp.float32),
    )


_layers = [_make_layer(i) for i in range(_NL)]


def kernel(x, edge_index, Wp, bp, W1, W2):
    row = edge_index[0]
    col = edge_index[1]
    pad = _EPAD - _E
    rowp = jnp.concatenate([row, jnp.zeros((pad,), row.dtype)])
    colp = jnp.concatenate([col, jnp.full((pad,), _ROWS, col.dtype)])

    bidx, bcnt, deg = _bucketize(rowp, colp)
    g, C, dis = _prologue(x, Wp, bp.reshape(1, _D), W2, deg[:_N])

    for i in range(_NL):
        spart = _propagate(bidx, bcnt, g)
        g = _layers[i](spart, g, C[i], dis, W1[i])
    return g
